# Initial kernel scaffold; baseline (speedup 1.0000x reference)
import jax
import jax.numpy as jnp
from jax.experimental import pallas as pl


def kernel(x, weight):
    raise NotImplementedError("write your pallas kernel here")


# speedup vs baseline: 1.5617x; 1.5617x over previous
import functools

import jax
import jax.numpy as jnp
from jax import lax
from jax.experimental import pallas as pl
from jax.experimental.pallas import tpu as pltpu
from jax.experimental.pallas import tpu_sc as plsc

_NUM_CORES = 2
_NUM_SUBCORES = 16
_NUM_WORKERS = _NUM_CORES * _NUM_SUBCORES


def _make_gather(batch: int, dim: int, chunk: int):
    assert batch % _NUM_WORKERS == 0
    b_per_w = batch // _NUM_WORKERS
    assert b_per_w % chunk == 0
    n_chunks = b_per_w // chunk

    mesh = plsc.VectorSubcoreMesh(core_axis_name="c", subcore_axis_name="s")

    @functools.partial(
        pl.kernel,
        mesh=mesh,
        out_type=jax.ShapeDtypeStruct((batch, dim), jnp.float32),
        scratch_types=[
            pltpu.VMEM((chunk,), jnp.int32),
            pltpu.VMEM((chunk, dim), jnp.float32),
            pltpu.SemaphoreType.DMA,
        ],
        compiler_params=pltpu.CompilerParams(use_tc_tiling_on_sc=False),
    )
    def emb(idx_hbm, table_hbm, out_hbm, idx_v, rows_v, sem):
        wid = lax.axis_index("s") * _NUM_CORES + lax.axis_index("c")
        base = wid * b_per_w

        def body(g, carry):
            off = base + g * chunk
            pltpu.sync_copy(idx_hbm.at[pl.ds(off, chunk)], idx_v)
            pltpu.async_copy(table_hbm.at[idx_v], rows_v, sem).wait()
            pltpu.sync_copy(rows_v, out_hbm.at[pl.ds(off, chunk)])
            return carry

        lax.fori_loop(0, n_chunks, body, 0)

    return emb


def kernel(x, weight):
    rows, cols = x.shape
    _, dim = weight.shape
    batch = rows * cols
    idx = x.reshape(batch).astype(jnp.int32)
    out = _make_gather(batch, dim, chunk=1664)(idx, weight)
    return out.reshape(rows, cols, dim)


# speedup vs baseline: 1.5765x; 1.0095x over previous
import functools

import jax
import jax.numpy as jnp
from jax import lax
from jax.experimental import pallas as pl
from jax.experimental.pallas import tpu as pltpu
from jax.experimental.pallas import tpu_sc as plsc

_NUM_CORES = 2
_NUM_SUBCORES = 16
_NUM_WORKERS = _NUM_CORES * _NUM_SUBCORES


def _make_gather(batch: int, dim: int, chunk: int):
    assert batch % _NUM_WORKERS == 0
    b_per_w = batch // _NUM_WORKERS
    assert b_per_w % chunk == 0
    n_chunks = b_per_w // chunk

    mesh = plsc.VectorSubcoreMesh(core_axis_name="c", subcore_axis_name="s")

    @functools.partial(
        pl.kernel,
        mesh=mesh,
        out_type=jax.ShapeDtypeStruct((batch, dim), jnp.float32),
        scratch_types=[
            pltpu.VMEM((b_per_w,), jnp.int32),
            pltpu.VMEM((2, chunk, dim), jnp.float32),
            pltpu.SemaphoreType.DMA((2,)),
            pltpu.SemaphoreType.DMA((2,)),
        ],
        compiler_params=pltpu.CompilerParams(use_tc_tiling_on_sc=False),
    )
    def emb(idx_hbm, table_hbm, out_hbm, idx_v, rows_v, gsem, wsem):
        wid = lax.axis_index("s") * _NUM_CORES + lax.axis_index("c")
        base = wid * b_per_w

        pltpu.sync_copy(idx_hbm.at[pl.ds(base, b_per_w)], idx_v)

        def gather_start(g):
            p = g % 2
            return pltpu.async_copy(
                table_hbm.at[idx_v.at[pl.ds(g * chunk, chunk)]],
                rows_v.at[p],
                gsem.at[p],
            )

        def write_start(g):
            p = g % 2
            return pltpu.async_copy(
                rows_v.at[p],
                out_hbm.at[pl.ds(base + g * chunk, chunk)],
                wsem.at[p],
            )

        gathers = {0: gather_start(0)}
        writes = {}
        for g in range(n_chunks):
            if g + 1 < n_chunks:
                if g - 1 >= 0:
                    writes[g - 1].wait()
                gathers[g + 1] = gather_start(g + 1)
            gathers[g].wait()
            writes[g] = write_start(g)
        writes[n_chunks - 2].wait()
        writes[n_chunks - 1].wait()

    return emb


def kernel(x, weight):
    rows, cols = x.shape
    _, dim = weight.shape
    batch = rows * cols
    idx = x.reshape(batch).astype(jnp.int32)
    out = _make_gather(batch, dim, chunk=1664)(idx, weight)
    return out.reshape(rows, cols, dim)


# speedup vs baseline: 1.6747x; 1.0623x over previous
import functools

import jax
import jax.numpy as jnp
from jax import lax
from jax.experimental import pallas as pl
from jax.experimental.pallas import tpu as pltpu
from jax.experimental.pallas import tpu_sc as plsc

_NUM_CORES = 2
_NUM_SUBCORES = 16
_NUM_WORKERS = _NUM_CORES * _NUM_SUBCORES


def _make_gather(batch: int, dim: int, chunk: int):
    assert batch % _NUM_WORKERS == 0
    b_per_w = batch // _NUM_WORKERS
    assert b_per_w % chunk == 0
    n_chunks = b_per_w // chunk

    mesh = plsc.VectorSubcoreMesh(core_axis_name="c", subcore_axis_name="s")

    @functools.partial(
        pl.kernel,
        mesh=mesh,
        out_type=jax.ShapeDtypeStruct((batch, dim), jnp.float32),
        scratch_types=[
            pltpu.VMEM((b_per_w,), jnp.int32),
            pltpu.VMEM((2, chunk, dim), jnp.float32),
            pltpu.SemaphoreType.DMA((2,)),
            pltpu.SemaphoreType.DMA((2,)),
        ],
        compiler_params=pltpu.CompilerParams(use_tc_tiling_on_sc=False),
    )
    def emb(idx_hbm, table_hbm, out_hbm, idx_v, rows_v, gsem, wsem):
        wid = lax.axis_index("s") * _NUM_CORES + lax.axis_index("c")
        base = wid * b_per_w

        pltpu.sync_copy(idx_hbm.at[pl.ds(base, b_per_w)], idx_v)

        def gather_start(g):
            p = g % 2
            return pltpu.async_copy(
                table_hbm.at[idx_v.at[pl.ds(g * chunk, chunk)]],
                rows_v.at[p],
                gsem.at[p],
            )

        def write_start(g):
            p = g % 2
            return pltpu.async_copy(
                rows_v.at[p],
                out_hbm.at[pl.ds(base + g * chunk, chunk)],
                wsem.at[p],
            )

        gathers = {0: gather_start(0)}
        writes = {}
        for g in range(n_chunks):
            if g + 1 < n_chunks:
                if g - 1 >= 0:
                    writes[g - 1].wait()
                gathers[g + 1] = gather_start(g + 1)
            gathers[g].wait()
            writes[g] = write_start(g)
        writes[n_chunks - 2].wait()
        writes[n_chunks - 1].wait()

    return emb


def kernel(x, weight):
    rows, cols = x.shape
    _, dim = weight.shape
    batch = rows * cols
    idx = x.T.reshape(batch).astype(jnp.int32)
    out = _make_gather(batch, dim, chunk=1664)(idx, weight)
    return out.reshape(cols, rows, dim).transpose(1, 0, 2)
